# Initial kernel scaffold; baseline (speedup 1.0000x reference)
import jax
import jax.numpy as jnp
from jax.experimental import pallas as pl


def kernel(x, Wg, Wd, bd, Wu, bu):
    raise NotImplementedError("write your pallas kernel here")


# speedup vs baseline: 10.4885x; 10.4885x over previous
import functools

import jax
import jax.numpy as jnp
from jax.experimental import pallas as pl

N = 8192
D_MODEL = 2048
BOTTLENECK = 64
NUM_EXPERTS = 8
EB = NUM_EXPERTS * BOTTLENECK

BN = 512


def _moe_body(x_ref, wgt_ref, wdt_ref, bdf_ref, wu_ref, bu_ref, o_ref):
    x = x_ref[...]

    logits = jnp.dot(x, wgt_ref[...], preferred_element_type=jnp.float32)
    mx = jnp.max(logits, axis=-1, keepdims=True)
    ex = jnp.exp(logits - mx)
    sm = ex / jnp.sum(ex, axis=-1, keepdims=True)

    col = jax.lax.broadcasted_iota(jnp.int32, (BN, NUM_EXPERTS), 1)
    m0 = jnp.max(sm, axis=-1, keepdims=True)
    e0 = jnp.min(jnp.where(sm == m0, col, NUM_EXPERTS), axis=-1, keepdims=True)
    oh0 = col == e0
    sm1 = jnp.where(oh0, -1.0, sm)
    m1 = jnp.max(sm1, axis=-1, keepdims=True)
    e1 = jnp.min(jnp.where(sm1 == m1, col, NUM_EXPERTS), axis=-1, keepdims=True)
    oh1 = col == e1
    denom = m0 + m1 + 1e-8
    m = jnp.where(oh0, m0 / denom, 0.0) + jnp.where(oh1, m1 / denom, 0.0)

    erow = jax.lax.broadcasted_iota(jnp.int32, (NUM_EXPERTS, EB), 0)
    ecol = jax.lax.broadcasted_iota(jnp.int32, (NUM_EXPERTS, EB), 1) // BOTTLENECK
    expand = (erow == ecol).astype(jnp.float32)
    m_exp = jnp.dot(m, expand, preferred_element_type=jnp.float32)

    down = jnp.dot(x, wdt_ref[...], preferred_element_type=jnp.float32) + bdf_ref[...]
    h = 0.5 * down * (1.0 + jax.lax.erf(down * 0.7071067811865476))
    g = h * m_exp
    out = jnp.dot(g, wu_ref[...], preferred_element_type=jnp.float32)
    out = out + jnp.dot(m, bu_ref[...], preferred_element_type=jnp.float32)
    o_ref[...] = out


@jax.jit
def kernel(x, Wg, Wd, bd, Wu, bu):
    wgt = Wg.T
    wdt = jnp.transpose(Wd, (2, 0, 1)).reshape(D_MODEL, EB)
    wu = jnp.transpose(Wu, (0, 2, 1)).reshape(EB, D_MODEL)
    bdf = bd.reshape(1, EB)

    grid = (N // BN,)
    return pl.pallas_call(
        _moe_body,
        grid=grid,
        in_specs=[
            pl.BlockSpec((BN, D_MODEL), lambda i: (i, 0)),
            pl.BlockSpec((D_MODEL, NUM_EXPERTS), lambda i: (0, 0)),
            pl.BlockSpec((D_MODEL, EB), lambda i: (0, 0)),
            pl.BlockSpec((1, EB), lambda i: (0, 0)),
            pl.BlockSpec((EB, D_MODEL), lambda i: (0, 0)),
            pl.BlockSpec((NUM_EXPERTS, D_MODEL), lambda i: (0, 0)),
        ],
        out_specs=pl.BlockSpec((BN, D_MODEL), lambda i: (i, 0)),
        out_shape=jax.ShapeDtypeStruct((N, D_MODEL), jnp.float32),
    )(x, wgt, wdt, bdf, wu, bu)


# speedup vs baseline: 10.8613x; 1.0355x over previous
import functools

import jax
import jax.numpy as jnp
from jax.experimental import pallas as pl

N = 8192
D_MODEL = 2048
BOTTLENECK = 64
NUM_EXPERTS = 8
EB = NUM_EXPERTS * BOTTLENECK

BN = 512


def _moe_body(x_ref, wgt_ref, wdt_ref, bdf_ref, wu_ref, bu_ref, o_ref):
    x = x_ref[...]

    logits = jnp.dot(x, wgt_ref[...], preferred_element_type=jnp.float32)
    mx = jnp.max(logits, axis=-1, keepdims=True)
    ex = jnp.exp(logits - mx)
    sm = ex / jnp.sum(ex, axis=-1, keepdims=True)

    col = jax.lax.broadcasted_iota(jnp.int32, (BN, NUM_EXPERTS), 1)
    m0 = jnp.max(sm, axis=-1, keepdims=True)
    e0 = jnp.min(jnp.where(sm == m0, col, NUM_EXPERTS), axis=-1, keepdims=True)
    oh0 = col == e0
    sm1 = jnp.where(oh0, -1.0, sm)
    m1 = jnp.max(sm1, axis=-1, keepdims=True)
    e1 = jnp.min(jnp.where(sm1 == m1, col, NUM_EXPERTS), axis=-1, keepdims=True)
    oh1 = col == e1
    denom = m0 + m1 + 1e-8
    m = jnp.where(oh0, m0 / denom, 0.0) + jnp.where(oh1, m1 / denom, 0.0)

    erow = jax.lax.broadcasted_iota(jnp.int32, (NUM_EXPERTS, EB), 0)
    ecol = jax.lax.broadcasted_iota(jnp.int32, (NUM_EXPERTS, EB), 1) // BOTTLENECK
    expand = (erow == ecol).astype(jnp.float32)
    m_exp = jnp.dot(m, expand, preferred_element_type=jnp.float32)

    xb = x.astype(jnp.bfloat16)
    down = jnp.dot(xb, wdt_ref[...], preferred_element_type=jnp.float32) + bdf_ref[...]
    h = 0.5 * down * (1.0 + jax.lax.erf(down * 0.7071067811865476))
    g = (h * m_exp).astype(jnp.bfloat16)
    out = jnp.dot(g, wu_ref[...], preferred_element_type=jnp.float32)
    out = out + jnp.dot(m, bu_ref[...], preferred_element_type=jnp.float32)
    o_ref[...] = out


@jax.jit
def kernel(x, Wg, Wd, bd, Wu, bu):
    wgt = Wg.T
    wdt = jnp.transpose(Wd, (2, 0, 1)).reshape(D_MODEL, EB).astype(jnp.bfloat16)
    wu = jnp.transpose(Wu, (0, 2, 1)).reshape(EB, D_MODEL).astype(jnp.bfloat16)
    bdf = bd.reshape(1, EB)

    grid = (N // BN,)
    return pl.pallas_call(
        _moe_body,
        grid=grid,
        in_specs=[
            pl.BlockSpec((BN, D_MODEL), lambda i: (i, 0)),
            pl.BlockSpec((D_MODEL, NUM_EXPERTS), lambda i: (0, 0)),
            pl.BlockSpec((D_MODEL, EB), lambda i: (0, 0)),
            pl.BlockSpec((1, EB), lambda i: (0, 0)),
            pl.BlockSpec((EB, D_MODEL), lambda i: (0, 0)),
            pl.BlockSpec((NUM_EXPERTS, D_MODEL), lambda i: (0, 0)),
        ],
        out_specs=pl.BlockSpec((BN, D_MODEL), lambda i: (i, 0)),
        out_shape=jax.ShapeDtypeStruct((N, D_MODEL), jnp.float32),
    )(x, wgt, wdt, bdf, wu, bu)
